# baseline (speedup 1.0000x reference)
import functools

import jax
import jax.numpy as jnp
from jax import lax
from jax.experimental import pallas as pl
from jax.experimental.pallas import tpu as pltpu
from jax.experimental.pallas import tpu_sc as plsc

NB_KNN = (10, 20, 100, 200)
MAX_K = 200
T = 0.07
NUM_CLASSES = 1000

Q = 1024
QB = 512
D = 128
N_TRAIN = 100000
N_PAD = 100352
BQ = 256
BN = 1024
NEG = -1e30
NGROUPS = N_PAD // 128
GM_W = 896
NCHUNKS = QB * NGROUPS

NC, NS = 2, 16
NW = NC * NS
QPW = QB // NW
NGBUD = 256
RB = 4
CAND = 512
KPAD = 256
CW = 1008
GMV = GM_W // 16

_SIGN = -2147483648
_MANT = 0x7FFFFFFF


def _iota16():
    return lax.broadcasted_iota(jnp.int32, (16,), 0)


def _splat(x):
    return jnp.full((16,), x, jnp.int32)


def _matmul_body(q_ref, t_ref, sim_ref, gm_ref):
    j = pl.program_id(1)
    s = lax.dot_general(
        q_ref[...], t_ref[...], (((1,), (1,)), ((), ())),
        preferred_element_type=jnp.float32,
    )
    col = j * BN + lax.broadcasted_iota(jnp.int32, (BQ, BN), 1)
    s = jnp.where(col < N_TRAIN, s, NEG)

    @pl.when(j % 16 == 0)
    def _():
        gm_ref[...] = jnp.full((BQ, 128), NEG, jnp.float32)

    lane = lax.broadcasted_iota(jnp.int32, (BQ, 128), 1)
    base = (j % 16) * 8
    cur = gm_ref[...]
    for g in range(8):
        sg = s[:, g * 128:(g + 1) * 128]
        sim_ref[:, g, :] = sg
        mg = jnp.max(sg, axis=1, keepdims=True)
        cur = jnp.where(lane == base + g, mg, cur)
    gm_ref[...] = cur


def _similarity(features_rank, tf_pad):
    return pl.pallas_call(
        _matmul_body,
        grid=(QB // BQ, N_PAD // BN),
        in_specs=[
            pl.BlockSpec((BQ, 128), lambda i, j: (i, 0)),
            pl.BlockSpec((BN, 128), lambda i, j: (j, 0)),
        ],
        out_specs=[
            pl.BlockSpec((BQ, 8, 128), lambda i, j: (i, j, 0)),
            pl.BlockSpec((BQ, 128), lambda i, j: (i, j // 16)),
        ],
        out_shape=[
            jax.ShapeDtypeStruct((QB, NGROUPS, 128), jnp.float32),
            jax.ShapeDtypeStruct((QB, GM_W), jnp.float32),
        ],
    )(features_rank, tf_pad)


def _select_body(gm_hbm, sim_hbm, outv_hbm, outi_hbm,
                 gmf_v, gmk_v, gl_v, cand_v, ov_v, oi_v, sem):
    wid = lax.axis_index("s") * NC + lax.axis_index("c")
    iota = _iota16()

    def row_body(r, _):
        q = wid * QPW + r
        pltpu.sync_copy(gm_hbm.at[q], gmf_v)

        def key_body(i, _):
            x = gmf_v[pl.ds(i * 16, 16)]
            b = plsc.bitcast(x, jnp.int32)
            gmk_v[pl.ds(i * 16, 16)] = b ^ (
                lax.shift_right_arithmetic(b, 31) & _MANT)
            return 0
        lax.fori_loop(0, GMV, key_body, 0)

        def bit_cond(state):
            _, cnt_b, it = state
            return jnp.logical_and(it < 32, jnp.max(cnt_b) > 240)

        def bit_body(state):
            t, cnt_b, it = state
            tryv = t + lax.shift_left(_splat(1), _splat(31 - it))

            def cnt_body(i, acc):
                a = acc
                for u in range(4):
                    m = gmk_v[pl.ds(i * 64 + u * 16, 16)] >= tryv
                    a = a + plsc.all_reduce_population_count(m)
                return a
            cnt = lax.fori_loop(0, GMV // 4, cnt_body, _splat(0))
            take = cnt >= MAX_K
            return (jnp.where(take, tryv, t), jnp.where(take, cnt, cnt_b),
                    it + 1)
        t, _, _ = lax.while_loop(
            bit_cond, bit_body,
            (jnp.full((16,), _SIGN, jnp.int32), _splat(GM_W), 0))
        tf = plsc.bitcast(jnp.where(t >= 0, t, t ^ _MANT), jnp.float32)

        pad_chunk = _splat(q * NGROUPS + 782) + (iota & 1)
        for h in range(2):
            for i in range(8):
                gl_v[h, pl.ds(i * 16, 16)] = pad_chunk

        def grp_body(i, off):
            m = gmk_v[pl.ds(i * 16, 16)] >= t
            pos = off + plsc.cumsum(jnp.where(m, 1, 0)) - 1
            m = m & (pos < NGBUD)
            chunk = _splat(q * NGROUPS) + _splat(i * 16) + iota
            plsc.store_scatter(
                gl_v, [lax.shift_right_logical(pos, 7), pos & 127],
                chunk, mask=m)
            return off + plsc.all_reduce_population_count(m)
        ng = lax.fori_loop(0, GMV, grp_body, _splat(0))
        ng_s = jnp.max(ng)

        cp0 = pltpu.async_copy(sim_hbm.at[gl_v.at[0]],
                               cand_v.at[pl.ds(0, 128)], sem)
        cp1 = pltpu.async_copy(sim_hbm.at[gl_v.at[1]],
                               cand_v.at[pl.ds(128, 128)], sem)
        cp0.wait()
        cp1.wait()

        def clr_body(i, _):
            ov_v[pl.ds(i * 16, 16)] = jnp.full((16,), NEG, jnp.float32)
            oi_v[pl.ds(i * 16, 16)] = _splat(0)
            return 0
        lax.fori_loop(0, CAND // 16, clr_body, 0)

        def el_body(ci, off):
            cis = _splat(ci)
            gabs = plsc.load_gather(
                gl_v, [lax.shift_right_logical(cis, 7), cis & 127])
            base_col = (gabs - q * NGROUPS) * 128
            vs, ms, cnts = [], [], []
            for j in range(8):
                v = cand_v[ci, pl.ds(j * 16, 16)]
                m = v >= tf
                vs.append(v)
                ms.append(m)
                cnts.append(plsc.all_reduce_population_count(m))
            bases = [off]
            for j in range(1, 8):
                bases.append(bases[-1] + cnts[j - 1])
            for j in range(8):
                pos = bases[j] + plsc.cumsum(jnp.where(ms[j], 1, 0)) - 1
                m = ms[j] & (pos < CAND)
                plsc.store_scatter(ov_v, [pos], vs[j], mask=m)
                plsc.store_scatter(oi_v, [pos], base_col + j * 16 + iota,
                                   mask=m)
            return bases[7] + cnts[7]
        lax.fori_loop(0, ng_s, el_body, _splat(0))

        pltpu.sync_copy(ov_v, outv_hbm.at[q])
        pltpu.sync_copy(oi_v, outi_hbm.at[q])
        return 0

    lax.fori_loop(0, QPW, row_body, 0)


def _select(gm, sim_chunks):
    mesh = plsc.VectorSubcoreMesh(
        core_axis_name="c", subcore_axis_name="s",
        num_cores=NC, num_subcores=NS)
    kern = pl.kernel(
        _select_body, mesh=mesh,
        out_type=(jax.ShapeDtypeStruct((QB, CAND), jnp.float32),
                  jax.ShapeDtypeStruct((QB, CAND), jnp.int32)),
        scratch_types=[
            pltpu.VMEM((GM_W,), jnp.float32),
            pltpu.VMEM((GM_W,), jnp.int32),
            pltpu.VMEM((2, 128), jnp.int32),
            pltpu.VMEM((NGBUD, 128), jnp.float32),
            pltpu.VMEM((CAND,), jnp.float32),
            pltpu.VMEM((CAND,), jnp.int32),
            pltpu.SemaphoreType.DMA,
        ],
        compiler_params=pltpu.CompilerParams(needs_layout_passes=False))
    return kern(gm, sim_chunks)


def _agg_body(tv_hbm, ti_hbm, lab_hbm, o10, o20, o100, o200,
              vvm, ivm, lvm, wbuf, buf, segaccB, sem):
    wid = lax.axis_index("s") * NC + lax.axis_index("c")
    iota = _iota16()
    outs = (o10, o20, o100, o200)

    def clr0(c, _):
        for r in range(16):
            buf[r, pl.ds(c * 16, 16)] = jnp.zeros((16,), jnp.float32)
        return 0
    lax.fori_loop(0, 4096 // 16, clr0, 0)

    def blk_body(bi, _):
        q0 = wid * QPW + bi * RB
        pltpu.sync_copy(tv_hbm.at[pl.ds(q0, RB)], vvm)
        pltpu.sync_copy(ti_hbm.at[pl.ds(2 * q0, 2 * RB)], ivm)
        cps = [pltpu.async_copy(lab_hbm.at[ivm.at[h]], lvm.at[h], sem)
               for h in range(2 * RB)]

        def clr1(c, _):
            z = jnp.zeros((16,), jnp.float32)
            for s in range(4):
                for r in range(RB):
                    segaccB[s, r, pl.ds(c * 16, 16)] = z
            return 0
        lax.fori_loop(0, CW // 16, clr1, 0)
        for cp in cps:
            cp.wait()

        for r in range(RB):
            vls = [vvm[r, pl.ds(j * 16, 16)] for j in range(16)]
            mx = vls[0]
            for j in range(1, 16):
                mx = jnp.maximum(mx, vls[j])
            mxs = jnp.full((16,), jnp.max(mx), jnp.float32)
            ssum = jnp.zeros((16,), jnp.float32)
            for j in range(16):
                e = jnp.exp((vls[j] - mxs) / T)
                wbuf[pl.ds(j * 16, 16)] = e
                ssum = ssum + e
            rec = jnp.ones((16,), jnp.float32) / jnp.full(
                (16,), jnp.sum(ssum), jnp.float32)

            idx2s = []
            for j in range(16):
                e = j * 16 + iota
                seg = (jnp.where(e >= 10, 1, 0) + jnp.where(e >= 20, 1, 0)
                       + jnp.where(e >= 100, 1, 0))
                lab = lvm[2 * r + j // 8, pl.ds((j % 8) * 16, 16)]
                idx2 = seg * 1024 + lab
                idx2s.append(idx2)
                plsc.addupdate_scatter(buf, [iota, idx2],
                                       wbuf[pl.ds(j * 16, 16)] * rec)

            for s in range(4):
                def red_body(c, _, s=s, r=r):
                    if s == 0:
                        acc = jnp.zeros((16,), jnp.float32)
                    else:
                        acc = segaccB[s - 1, r, pl.ds(c * 16, 16)]
                    for rr in range(16):
                        acc = acc + buf[rr, pl.ds(s * 1024 + c * 16, 16)]
                    segaccB[s, r, pl.ds(c * 16, 16)] = acc
                    return 0
                lax.fori_loop(0, CW // 16, red_body, 0)

            for j in range(16):
                plsc.store_scatter(buf, [iota, idx2s[j]],
                                   jnp.zeros((16,), jnp.float32))

        for s in range(4):
            pltpu.sync_copy(segaccB.at[s], outs[s].at[pl.ds(q0, RB)])
        return 0

    lax.fori_loop(0, QPW // RB, blk_body, 0)


def _aggregate(top_v, top_i, train_labels):
    mesh = plsc.VectorSubcoreMesh(
        core_axis_name="c", subcore_axis_name="s",
        num_cores=NC, num_subcores=NS)
    out = jax.ShapeDtypeStruct((QB, CW), jnp.float32)
    kern = pl.kernel(
        _agg_body, mesh=mesh,
        out_type=(out, out, out, out),
        scratch_types=[
            pltpu.VMEM((RB, KPAD), jnp.float32),
            pltpu.VMEM((2 * RB, 128), jnp.int32),
            pltpu.VMEM((2 * RB, 128), jnp.int32),
            pltpu.VMEM((KPAD,), jnp.float32),
            pltpu.VMEM((16, 4096), jnp.float32),
            pltpu.VMEM((4, RB, CW), jnp.float32),
            pltpu.SemaphoreType.DMA,
        ],
        compiler_params=pltpu.CompilerParams(needs_layout_passes=False))
    return kern(top_v, top_i, train_labels)


def kernel(features_rank, train_features, train_labels):
    tf_pad = jnp.pad(train_features, ((0, N_PAD - N_TRAIN), (0, 0)))
    halves = []
    for h in range(Q // QB):
        fr = lax.slice_in_dim(features_rank, h * QB, (h + 1) * QB)
        sim3, gm = _similarity(fr, tf_pad)
        cand_v, cand_i = _select(gm, sim3.reshape(NCHUNKS, 128))
        top_v, pos = lax.top_k(cand_v, MAX_K)
        top_i = jnp.take_along_axis(cand_i, pos, axis=1)
        tvp = jnp.pad(top_v, ((0, 0), (0, KPAD - MAX_K)),
                      constant_values=NEG)
        tip = jnp.pad(top_i, ((0, 0), (0, KPAD - MAX_K)))
        halves.append(
            _aggregate(tvp, tip.reshape(QB * 2, 128), train_labels))
    return tuple(
        jnp.concatenate([halves[h][s][:, :NUM_CLASSES]
                         for h in range(Q // QB)], axis=0)
        for s in range(4))


# speedup vs baseline: 1.0728x; 1.0728x over previous
import functools

import jax
import jax.numpy as jnp
from jax import lax
from jax.experimental import pallas as pl
from jax.experimental.pallas import tpu as pltpu
from jax.experimental.pallas import tpu_sc as plsc

NB_KNN = (10, 20, 100, 200)
MAX_K = 200
T = 0.07
NUM_CLASSES = 1000

Q = 1024
QB = 512
D = 128
N_TRAIN = 100000
N_PAD = 100352
BQ = 256
BN = 1024
NEG = -1e30
NGROUPS = N_PAD // 128
GM_W = 896
NCHUNKS = QB * NGROUPS

NC, NS = 2, 16
NW = NC * NS
QPW = QB // NW
NGBUD = 256
RB = 4
CAND = 512
KPAD = 256
CW = 1008
GMV = GM_W // 16

_SIGN = -2147483648
_MANT = 0x7FFFFFFF


def _iota16():
    return lax.broadcasted_iota(jnp.int32, (16,), 0)


def _splat(x):
    return jnp.full((16,), x, jnp.int32)


def _matmul_body(q_ref, t_ref, sim_ref, gm_ref):
    j = pl.program_id(1)
    s = lax.dot_general(
        q_ref[...], t_ref[...], (((1,), (1,)), ((), ())),
        preferred_element_type=jnp.float32,
    )
    col = j * BN + lax.broadcasted_iota(jnp.int32, (BQ, BN), 1)
    s = jnp.where(col < N_TRAIN, s, NEG)

    @pl.when(j % 16 == 0)
    def _():
        gm_ref[...] = jnp.full((BQ, 128), NEG, jnp.float32)

    lane = lax.broadcasted_iota(jnp.int32, (BQ, 128), 1)
    base = (j % 16) * 8
    cur = gm_ref[...]
    for g in range(8):
        sg = s[:, g * 128:(g + 1) * 128]
        sim_ref[:, g, :] = sg
        mg = jnp.max(sg, axis=1, keepdims=True)
        cur = jnp.where(lane == base + g, mg, cur)
    gm_ref[...] = cur


def _similarity(features_rank, tf_pad):
    return pl.pallas_call(
        _matmul_body,
        grid=(QB // BQ, N_PAD // BN),
        in_specs=[
            pl.BlockSpec((BQ, 128), lambda i, j: (i, 0)),
            pl.BlockSpec((BN, 128), lambda i, j: (j, 0)),
        ],
        out_specs=[
            pl.BlockSpec((BQ, 8, 128), lambda i, j: (i, j, 0)),
            pl.BlockSpec((BQ, 128), lambda i, j: (i, j // 16)),
        ],
        out_shape=[
            jax.ShapeDtypeStruct((QB, NGROUPS, 128), jnp.float32),
            jax.ShapeDtypeStruct((QB, GM_W), jnp.float32),
        ],
    )(features_rank, tf_pad)


def _select_body(gm_hbm, sim_hbm, outv_hbm, outi_hbm,
                 gmf_v, gmk_v, gl_v, cand_v, ov_v, oi_v, sem):
    wid = lax.axis_index("s") * NC + lax.axis_index("c")
    iota = _iota16()

    def row_body(r, _):
        q = wid * QPW + r
        pltpu.sync_copy(gm_hbm.at[q], gmf_v)

        def key_body(i, _):
            x = gmf_v[pl.ds(i * 16, 16)]
            b = plsc.bitcast(x, jnp.int32)
            gmk_v[pl.ds(i * 16, 16)] = b ^ (
                lax.shift_right_arithmetic(b, 31) & _MANT)
            return 0
        lax.fori_loop(0, GMV, key_body, 0)

        def bit_cond(state):
            _, cnt_b, it = state
            return jnp.logical_and(it < 32, jnp.max(cnt_b) > 240)

        def bit_body(state):
            t, cnt_b, it = state
            tryv = t + lax.shift_left(_splat(1), _splat(31 - it))

            def cnt_body(i, acc):
                a = acc
                for u in range(4):
                    m = gmk_v[pl.ds(i * 64 + u * 16, 16)] >= tryv
                    a = a + plsc.all_reduce_population_count(m)
                return a
            cnt = lax.fori_loop(0, GMV // 4, cnt_body, _splat(0))
            take = cnt >= MAX_K
            return (jnp.where(take, tryv, t), jnp.where(take, cnt, cnt_b),
                    it + 1)
        t, _, _ = lax.while_loop(
            bit_cond, bit_body,
            (jnp.full((16,), _SIGN, jnp.int32), _splat(GM_W), 0))
        tf = plsc.bitcast(jnp.where(t >= 0, t, t ^ _MANT), jnp.float32)

        pad_chunk = _splat(q * NGROUPS + 782) + (iota & 1)
        for h in range(2):
            for i in range(8):
                gl_v[h, pl.ds(i * 16, 16)] = pad_chunk

        def grp_body(i, off):
            m = gmk_v[pl.ds(i * 16, 16)] >= t
            pos = off + plsc.cumsum(jnp.where(m, 1, 0)) - 1
            m = m & (pos < NGBUD)
            chunk = _splat(q * NGROUPS) + _splat(i * 16) + iota
            plsc.store_scatter(
                gl_v, [lax.shift_right_logical(pos, 7), pos & 127],
                chunk, mask=m)
            return off + plsc.all_reduce_population_count(m)
        ng = lax.fori_loop(0, GMV, grp_body, _splat(0))
        ng_s = jnp.max(ng)

        cp0 = pltpu.async_copy(sim_hbm.at[gl_v.at[0]],
                               cand_v.at[pl.ds(0, 128)], sem)
        cp1 = pltpu.async_copy(sim_hbm.at[gl_v.at[1]],
                               cand_v.at[pl.ds(128, 128)], sem)
        cp0.wait()
        cp1.wait()

        def clr_body(i, _):
            ov_v[pl.ds(i * 16, 16)] = jnp.full((16,), NEG, jnp.float32)
            oi_v[pl.ds(i * 16, 16)] = _splat(0)
            return 0
        lax.fori_loop(0, CAND // 16, clr_body, 0)

        def el_body(ci, off):
            cis = _splat(ci)
            gabs = plsc.load_gather(
                gl_v, [lax.shift_right_logical(cis, 7), cis & 127])
            base_col = (gabs - q * NGROUPS) * 128
            vs, ms, cnts = [], [], []
            for j in range(8):
                v = cand_v[ci, pl.ds(j * 16, 16)]
                m = v >= tf
                vs.append(v)
                ms.append(m)
                cnts.append(plsc.all_reduce_population_count(m))
            bases = [off]
            for j in range(1, 8):
                bases.append(bases[-1] + cnts[j - 1])
            for j in range(8):
                pos = bases[j] + plsc.cumsum(jnp.where(ms[j], 1, 0)) - 1
                m = ms[j] & (pos < CAND)
                plsc.store_scatter(ov_v, [pos], vs[j], mask=m)
                plsc.store_scatter(oi_v, [pos], base_col + j * 16 + iota,
                                   mask=m)
            return bases[7] + cnts[7]
        lax.fori_loop(0, ng_s, el_body, _splat(0))

        pltpu.sync_copy(ov_v, outv_hbm.at[q])
        pltpu.sync_copy(oi_v, outi_hbm.at[q])
        return 0

    lax.fori_loop(0, QPW, row_body, 0)


def _select(gm, sim_chunks):
    mesh = plsc.VectorSubcoreMesh(
        core_axis_name="c", subcore_axis_name="s",
        num_cores=NC, num_subcores=NS)
    kern = pl.kernel(
        _select_body, mesh=mesh,
        out_type=(jax.ShapeDtypeStruct((QB, CAND), jnp.float32),
                  jax.ShapeDtypeStruct((QB, CAND), jnp.int32)),
        scratch_types=[
            pltpu.VMEM((GM_W,), jnp.float32),
            pltpu.VMEM((GM_W,), jnp.int32),
            pltpu.VMEM((2, 128), jnp.int32),
            pltpu.VMEM((NGBUD, 128), jnp.float32),
            pltpu.VMEM((CAND,), jnp.float32),
            pltpu.VMEM((CAND,), jnp.int32),
            pltpu.SemaphoreType.DMA,
        ],
        compiler_params=pltpu.CompilerParams(needs_layout_passes=False))
    return kern(gm, sim_chunks)


def _agg_body(tv_hbm, ti_hbm, lab_hbm, o10, o20, o100, o200,
              vvm, ivm, lvm, wbuf, buf, segacc, sem):
    wid = lax.axis_index("s") * NC + lax.axis_index("c")
    iota = _iota16()
    outs = (o10, o20, o100, o200)

    def clr0(c, _):
        for r in range(16):
            buf[r, pl.ds(c * 16, 16)] = jnp.zeros((16,), jnp.float32)
        return 0
    lax.fori_loop(0, 4096 // 16, clr0, 0)

    def row_body(r, _):
        q = wid * QPW + r
        pltpu.sync_copy(tv_hbm.at[q], vvm)
        pltpu.sync_copy(ti_hbm.at[q], ivm)
        cp0 = pltpu.async_copy(lab_hbm.at[ivm.at[0]], lvm.at[0], sem)
        cp1 = pltpu.async_copy(lab_hbm.at[ivm.at[1]], lvm.at[1], sem)

        vls = [vvm[pl.ds(j * 16, 16)] for j in range(16)]
        mx = vls[0]
        for j in range(1, 16):
            mx = jnp.maximum(mx, vls[j])
        mxs = jnp.full((16,), jnp.max(mx), jnp.float32)
        ssum = jnp.zeros((16,), jnp.float32)
        for j in range(16):
            e = jnp.exp((vls[j] - mxs) / T)
            wbuf[pl.ds(j * 16, 16)] = e
            ssum = ssum + e
        rec = jnp.ones((16,), jnp.float32) / jnp.full(
            (16,), jnp.sum(ssum), jnp.float32)

        cp0.wait()
        cp1.wait()

        idx2s = []
        for j in range(16):
            e = j * 16 + iota
            seg = (jnp.where(e >= 10, 1, 0) + jnp.where(e >= 20, 1, 0)
                   + jnp.where(e >= 100, 1, 0))
            lab = lvm[j // 8, pl.ds((j % 8) * 16, 16)]
            idx2 = seg * 1024 + lab
            idx2s.append(idx2)
            plsc.addupdate_scatter(buf, [iota, idx2],
                                   wbuf[pl.ds(j * 16, 16)] * rec)

        def clr1(c, _):
            segacc[pl.ds(c * 16, 16)] = jnp.zeros((16,), jnp.float32)
            return 0
        lax.fori_loop(0, CW // 16, clr1, 0)
        for s in range(4):
            def red_body(c, _, s=s):
                acc = segacc[pl.ds(c * 16, 16)]
                for rr in range(16):
                    acc = acc + buf[rr, pl.ds(s * 1024 + c * 16, 16)]
                segacc[pl.ds(c * 16, 16)] = acc
                return 0
            lax.fori_loop(0, CW // 16, red_body, 0)
            pltpu.sync_copy(segacc, outs[s].at[q])

        for j in range(16):
            plsc.store_scatter(buf, [iota, idx2s[j]],
                               jnp.zeros((16,), jnp.float32))
        return 0

    lax.fori_loop(0, QPW, row_body, 0)


def _aggregate(top_v, top_i, train_labels):
    mesh = plsc.VectorSubcoreMesh(
        core_axis_name="c", subcore_axis_name="s",
        num_cores=NC, num_subcores=NS)
    out = jax.ShapeDtypeStruct((QB, CW), jnp.float32)
    kern = pl.kernel(
        _agg_body, mesh=mesh,
        out_type=(out, out, out, out),
        scratch_types=[
            pltpu.VMEM((KPAD,), jnp.float32),
            pltpu.VMEM((2, 128), jnp.int32),
            pltpu.VMEM((2, 128), jnp.int32),
            pltpu.VMEM((KPAD,), jnp.float32),
            pltpu.VMEM((16, 4096), jnp.float32),
            pltpu.VMEM((CW,), jnp.float32),
            pltpu.SemaphoreType.DMA,
        ],
        compiler_params=pltpu.CompilerParams(needs_layout_passes=False))
    return kern(top_v, top_i, train_labels)


def kernel(features_rank, train_features, train_labels):
    tf_pad = jnp.pad(train_features, ((0, N_PAD - N_TRAIN), (0, 0)))
    halves = []
    for h in range(Q // QB):
        fr = lax.slice_in_dim(features_rank, h * QB, (h + 1) * QB)
        sim3, gm = _similarity(fr, tf_pad)
        cand_v, cand_i = _select(gm, sim3.reshape(NCHUNKS, 128))
        top_v, pos = lax.top_k(cand_v, MAX_K)
        top_i = jnp.take_along_axis(cand_i, pos, axis=1)
        tvp = jnp.pad(top_v, ((0, 0), (0, KPAD - MAX_K)),
                      constant_values=NEG)
        tip = jnp.pad(top_i, ((0, 0), (0, KPAD - MAX_K)))
        halves.append(
            _aggregate(tvp, tip.reshape(QB, 2, 128), train_labels))
    return tuple(
        jnp.concatenate([halves[h][s][:, :NUM_CLASSES]
                         for h in range(Q // QB)], axis=0)
        for s in range(4))


# speedup vs baseline: 1.0735x; 1.0007x over previous
import functools

import jax
import jax.numpy as jnp
from jax import lax
from jax.experimental import pallas as pl
from jax.experimental.pallas import tpu as pltpu
from jax.experimental.pallas import tpu_sc as plsc

NB_KNN = (10, 20, 100, 200)
MAX_K = 200
T = 0.07
NUM_CLASSES = 1000

Q = 1024
QB = 512
D = 128
N_TRAIN = 100000
N_PAD = 100352
BQ = 256
BN = 1024
NEG = -1e30
NGROUPS = N_PAD // 128
GM_W = 896
NCHUNKS = QB * NGROUPS

NC, NS = 2, 16
NW = NC * NS
QPW = QB // NW
NGBUD = 256
CAND = 512
KPAD = 256
CW = 1008
GMV = GM_W // 16

_SIGN = -2147483648
_MANT = 0x7FFFFFFF


def _iota16():
    return lax.broadcasted_iota(jnp.int32, (16,), 0)


def _splat(x):
    return jnp.full((16,), x, jnp.int32)


def _matmul_body(q_ref, t_ref, sim_ref, gm_ref):
    j = pl.program_id(1)
    s = lax.dot_general(
        q_ref[...], t_ref[...], (((1,), (1,)), ((), ())),
        preferred_element_type=jnp.float32,
    )
    col = j * BN + lax.broadcasted_iota(jnp.int32, (BQ, BN), 1)
    s = jnp.where(col < N_TRAIN, s, NEG)

    @pl.when(j % 16 == 0)
    def _():
        gm_ref[...] = jnp.full((BQ, 128), NEG, jnp.float32)

    lane = lax.broadcasted_iota(jnp.int32, (BQ, 128), 1)
    base = (j % 16) * 8
    cur = gm_ref[...]
    for g in range(8):
        sg = s[:, g * 128:(g + 1) * 128]
        sim_ref[:, g, :] = sg
        mg = jnp.max(sg, axis=1, keepdims=True)
        cur = jnp.where(lane == base + g, mg, cur)
    gm_ref[...] = cur


def _similarity(features_rank, tf_pad):
    return pl.pallas_call(
        _matmul_body,
        grid=(QB // BQ, N_PAD // BN),
        in_specs=[
            pl.BlockSpec((BQ, 128), lambda i, j: (i, 0)),
            pl.BlockSpec((BN, 128), lambda i, j: (j, 0)),
        ],
        out_specs=[
            pl.BlockSpec((BQ, 8, 128), lambda i, j: (i, j, 0)),
            pl.BlockSpec((BQ, 128), lambda i, j: (i, j // 16)),
        ],
        out_shape=[
            jax.ShapeDtypeStruct((QB, NGROUPS, 128), jnp.float32),
            jax.ShapeDtypeStruct((QB, GM_W), jnp.float32),
        ],
    )(features_rank, tf_pad)


def _select_body(gm_hbm, sim_hbm, outv_hbm, outi_hbm,
                 gmf_v, gmk_v, gl_v, cand_v, ov_v, oi_v, sem):
    wid = lax.axis_index("s") * NC + lax.axis_index("c")
    iota = _iota16()

    def row_body(r, _):
        q = wid * QPW + r
        pltpu.sync_copy(gm_hbm.at[q], gmf_v)

        def key_body(i, _):
            x = gmf_v[pl.ds(i * 16, 16)]
            b = plsc.bitcast(x, jnp.int32)
            gmk_v[pl.ds(i * 16, 16)] = b ^ (
                lax.shift_right_arithmetic(b, 31) & _MANT)
            return 0
        lax.fori_loop(0, GMV, key_body, 0)

        def bit_cond(state):
            _, cnt_b, it = state
            return jnp.logical_and(it < 32, jnp.max(cnt_b) > 240)

        def bit_body(state):
            t, cnt_b, it = state
            tryv = t + lax.shift_left(_splat(1), _splat(31 - it))

            def cnt_body(i, acc):
                a = acc
                for u in range(4):
                    m = gmk_v[pl.ds(i * 64 + u * 16, 16)] >= tryv
                    a = a + plsc.all_reduce_population_count(m)
                return a
            cnt = lax.fori_loop(0, GMV // 4, cnt_body, _splat(0))
            take = cnt >= MAX_K
            return (jnp.where(take, tryv, t), jnp.where(take, cnt, cnt_b),
                    it + 1)
        t, _, _ = lax.while_loop(
            bit_cond, bit_body,
            (jnp.full((16,), _SIGN, jnp.int32), _splat(GM_W), 0))
        tf = plsc.bitcast(jnp.where(t >= 0, t, t ^ _MANT), jnp.float32)

        pad_chunk = _splat(q * NGROUPS + 782) + (iota & 1)
        for h in range(2):
            for i in range(8):
                gl_v[h, pl.ds(i * 16, 16)] = pad_chunk

        def grp_body(i, off):
            m = gmk_v[pl.ds(i * 16, 16)] >= t
            pos = off + plsc.cumsum(jnp.where(m, 1, 0)) - 1
            m = m & (pos < NGBUD)
            chunk = _splat(q * NGROUPS) + _splat(i * 16) + iota
            plsc.store_scatter(
                gl_v, [lax.shift_right_logical(pos, 7), pos & 127],
                chunk, mask=m)
            return off + plsc.all_reduce_population_count(m)
        ng = lax.fori_loop(0, GMV, grp_body, _splat(0))
        ng_s = jnp.max(ng)

        cp0 = pltpu.async_copy(sim_hbm.at[gl_v.at[0]],
                               cand_v.at[pl.ds(0, 128)], sem)
        cp1 = pltpu.async_copy(sim_hbm.at[gl_v.at[1]],
                               cand_v.at[pl.ds(128, 128)], sem)
        cp0.wait()
        cp1.wait()

        def clr_body(i, _):
            ov_v[pl.ds(i * 16, 16)] = jnp.full((16,), NEG, jnp.float32)
            oi_v[pl.ds(i * 16, 16)] = _splat(0)
            return 0
        lax.fori_loop(0, CAND // 16, clr_body, 0)

        def el_body(ci, off):
            cis = _splat(ci)
            gabs = plsc.load_gather(
                gl_v, [lax.shift_right_logical(cis, 7), cis & 127])
            base_col = (gabs - q * NGROUPS) * 128
            vs, ms, cnts = [], [], []
            for j in range(8):
                v = cand_v[ci, pl.ds(j * 16, 16)]
                m = v >= tf
                vs.append(v)
                ms.append(m)
                cnts.append(plsc.all_reduce_population_count(m))
            bases = [off]
            for j in range(1, 8):
                bases.append(bases[-1] + cnts[j - 1])
            for j in range(8):
                pos = bases[j] + plsc.cumsum(jnp.where(ms[j], 1, 0)) - 1
                m = ms[j] & (pos < CAND)
                plsc.store_scatter(ov_v, [pos], vs[j], mask=m)
                plsc.store_scatter(oi_v, [pos], base_col + j * 16 + iota,
                                   mask=m)
            return bases[7] + cnts[7]
        lax.fori_loop(0, ng_s, el_body, _splat(0))

        pltpu.sync_copy(ov_v, outv_hbm.at[q])
        pltpu.sync_copy(oi_v, outi_hbm.at[q])
        return 0

    lax.fori_loop(0, QPW, row_body, 0)


def _select(gm, sim_chunks):
    mesh = plsc.VectorSubcoreMesh(
        core_axis_name="c", subcore_axis_name="s",
        num_cores=NC, num_subcores=NS)
    kern = pl.kernel(
        _select_body, mesh=mesh,
        out_type=(jax.ShapeDtypeStruct((QB, CAND), jnp.float32),
                  jax.ShapeDtypeStruct((QB, CAND), jnp.int32)),
        scratch_types=[
            pltpu.VMEM((GM_W,), jnp.float32),
            pltpu.VMEM((GM_W,), jnp.int32),
            pltpu.VMEM((2, 128), jnp.int32),
            pltpu.VMEM((NGBUD, 128), jnp.float32),
            pltpu.VMEM((CAND,), jnp.float32),
            pltpu.VMEM((CAND,), jnp.int32),
            pltpu.SemaphoreType.DMA,
        ],
        compiler_params=pltpu.CompilerParams(needs_layout_passes=False))
    return kern(gm, sim_chunks)


def _agg_body(tv_hbm, ti_hbm, lab_hbm, o10, o20, o100, o200,
              vvm, ivm, lvm, wbuf, buf, segacc, sem):
    wid = lax.axis_index("s") * NC + lax.axis_index("c")
    iota = _iota16()
    outs = (o10, o20, o100, o200)

    def clr0(c, _):
        for r in range(16):
            buf[r, pl.ds(c * 16, 16)] = jnp.zeros((16,), jnp.float32)
        return 0
    lax.fori_loop(0, 4096 // 16, clr0, 0)

    def row_body(r, _):
        q = wid * QPW + r
        pltpu.sync_copy(tv_hbm.at[q], vvm)
        pltpu.sync_copy(ti_hbm.at[q], ivm)
        cp0 = pltpu.async_copy(lab_hbm.at[ivm.at[0]], lvm.at[0], sem)
        cp1 = pltpu.async_copy(lab_hbm.at[ivm.at[1]], lvm.at[1], sem)

        vls = [vvm[pl.ds(j * 16, 16)] for j in range(16)]
        mx = vls[0]
        for j in range(1, 16):
            mx = jnp.maximum(mx, vls[j])
        mxs = jnp.full((16,), jnp.max(mx), jnp.float32)
        ssum = jnp.zeros((16,), jnp.float32)
        for j in range(16):
            e = jnp.exp((vls[j] - mxs) / T)
            wbuf[pl.ds(j * 16, 16)] = e
            ssum = ssum + e
        rec = jnp.ones((16,), jnp.float32) / jnp.full(
            (16,), jnp.sum(ssum), jnp.float32)

        cp0.wait()
        cp1.wait()

        idx2s = []
        for j in range(16):
            e = j * 16 + iota
            seg = (jnp.where(e >= 10, 1, 0) + jnp.where(e >= 20, 1, 0)
                   + jnp.where(e >= 100, 1, 0))
            lab = lvm[j // 8, pl.ds((j % 8) * 16, 16)]
            idx2 = seg * 1024 + lab
            idx2s.append(idx2)
            plsc.addupdate_scatter(buf, [iota, idx2],
                                   wbuf[pl.ds(j * 16, 16)] * rec)

        def clr1(c, _):
            segacc[pl.ds(c * 16, 16)] = jnp.zeros((16,), jnp.float32)
            return 0
        lax.fori_loop(0, CW // 16, clr1, 0)
        for s in range(4):
            def red_body(c, _, s=s):
                acc = segacc[pl.ds(c * 16, 16)]
                for rr in range(16):
                    acc = acc + buf[rr, pl.ds(s * 1024 + c * 16, 16)]
                segacc[pl.ds(c * 16, 16)] = acc
                return 0
            lax.fori_loop(0, CW // 16, red_body, 0)
            pltpu.sync_copy(segacc, outs[s].at[q])

        for j in range(16):
            plsc.store_scatter(buf, [iota, idx2s[j]],
                               jnp.zeros((16,), jnp.float32))
        return 0

    lax.fori_loop(0, QPW, row_body, 0)


def _aggregate(top_v, top_i, train_labels):
    mesh = plsc.VectorSubcoreMesh(
        core_axis_name="c", subcore_axis_name="s",
        num_cores=NC, num_subcores=NS)
    out = jax.ShapeDtypeStruct((QB, CW), jnp.float32)
    kern = pl.kernel(
        _agg_body, mesh=mesh,
        out_type=(out, out, out, out),
        scratch_types=[
            pltpu.VMEM((KPAD,), jnp.float32),
            pltpu.VMEM((2, 128), jnp.int32),
            pltpu.VMEM((2, 128), jnp.int32),
            pltpu.VMEM((KPAD,), jnp.float32),
            pltpu.VMEM((16, 4096), jnp.float32),
            pltpu.VMEM((CW,), jnp.float32),
            pltpu.SemaphoreType.DMA,
        ],
        compiler_params=pltpu.CompilerParams(needs_layout_passes=False))
    return kern(top_v, top_i, train_labels)


def kernel(features_rank, train_features, train_labels):
    tf_pad = jnp.pad(train_features, ((0, N_PAD - N_TRAIN), (0, 0)))
    halves = []
    for h in range(Q // QB):
        fr = lax.slice_in_dim(features_rank, h * QB, (h + 1) * QB)
        sim3, gm = _similarity(fr, tf_pad)
        cand_v, cand_i = _select(gm, sim3.reshape(NCHUNKS, 128))
        top_v, pos = lax.top_k(cand_v, MAX_K)
        top_i = jnp.take_along_axis(cand_i, pos, axis=1)
        tvp = jnp.pad(top_v, ((0, 0), (0, KPAD - MAX_K)),
                      constant_values=NEG)
        tip = jnp.pad(top_i, ((0, 0), (0, KPAD - MAX_K)))
        halves.append(
            _aggregate(tvp, tip.reshape(QB, 2, 128), train_labels))
    return tuple(
        jnp.concatenate([halves[h][s][:, :NUM_CLASSES]
                         for h in range(Q // QB)], axis=0)
        for s in range(4))
